# baseline (speedup 1.0000x reference)
import functools

import jax
import jax.numpy as jnp
from jax import lax
from jax.experimental import pallas as pl
from jax.experimental.pallas import tpu as pltpu
from jax.experimental.pallas import tpu_sc as plsc

VOCAB = 100000
SEQ_LEN = 200
BATCH = 1024
NTOK = SEQ_LEN * BATCH

TOKEN_WAYS = 8
VOCAB_WAYS = 4
VOCAB_PAD = 102400
BINS_PER_TILE = VOCAB_PAD // VOCAB_WAYS
TOK_PER_TILE = NTOK // TOKEN_WAYS

CHUNK = 6400
NCHUNK = TOK_PER_TILE // CHUNK
VREGS_PER_CHUNK = CHUNK // 16

NSUB = 16
SLICE = VOCAB_PAD // NSUB
ROWS_PER_TILE = BATCH // 32


def _sc_histogram(src_flat):
    mesh = plsc.VectorSubcoreMesh(core_axis_name="c", subcore_axis_name="s")

    @functools.partial(
        pl.kernel,
        mesh=mesh,
        out_type=jax.ShapeDtypeStruct((TOKEN_WAYS * VOCAB_PAD,), jnp.float32),
        compiler_params=pltpu.CompilerParams(needs_layout_passes=False),
        scratch_types=[
            pltpu.VMEM((CHUNK,), jnp.int32),
            pltpu.VMEM((CHUNK,), jnp.int32),
            pltpu.VMEM((BINS_PER_TILE,), jnp.float32),
            pltpu.SemaphoreType.DMA,
            pltpu.SemaphoreType.DMA,
        ],
    )
    def hist_kernel(src_hbm, out_hbm, buf0, buf1, hist, sem0, sem1):
        c = lax.axis_index("c")
        s = lax.axis_index("s")
        wid = s * 2 + c
        g = wid // VOCAB_WAYS
        v = wid % VOCAB_WAYS
        base = v * BINS_PER_TILE
        tok0 = g * TOK_PER_TILE

        zeros16 = jnp.zeros((16,), jnp.float32)

        def zero_body(i, carry):
            hist[pl.ds(i * 16, 16)] = zeros16
            return carry

        lax.fori_loop(0, BINS_PER_TILE // 16, zero_body, 0)

        ones16 = jnp.ones((16,), jnp.float32)
        bufs = (buf0, buf1)
        sems = (sem0, sem1)

        copies = [None, None]
        copies[0] = pltpu.async_copy(
            src_hbm.at[pl.ds(tok0, CHUNK)], buf0, sem0)
        for ci in range(NCHUNK):
            if ci + 1 < NCHUNK:
                copies[(ci + 1) % 2] = pltpu.async_copy(
                    src_hbm.at[pl.ds(tok0 + (ci + 1) * CHUNK, CHUNK)],
                    bufs[(ci + 1) % 2],
                    sems[(ci + 1) % 2],
                )
            copies[ci % 2].wait()
            buf = bufs[ci % 2]

            def body(i, carry):
                tok = buf[pl.ds(i * 16, 16)]
                rel = tok - base
                mask = (rel >= 0) & (rel < BINS_PER_TILE)
                plsc.addupdate_scatter(hist, [rel], ones16, mask=mask)
                return carry

            lax.fori_loop(0, VREGS_PER_CHUNK, body, 0)

        pltpu.sync_copy(
            hist, out_hbm.at[pl.ds(g * VOCAB_PAD + base, BINS_PER_TILE)])

    return hist_kernel(src_flat)


CH_B = 1600
NCH_B = VOCAB_PAD // CH_B


def _sc_broadcast(partials, ign16):
    mesh = plsc.VectorSubcoreMesh(core_axis_name="c", subcore_axis_name="s")

    @functools.partial(
        pl.kernel,
        mesh=mesh,
        out_type=jax.ShapeDtypeStruct((BATCH * VOCAB,), jnp.float32),
        compiler_params=pltpu.CompilerParams(needs_layout_passes=False),
        scratch_types=[
            pltpu.VMEM((TOKEN_WAYS * CH_B,), jnp.float32),
            pltpu.VMEM((VOCAB_PAD,), jnp.float32),
            pltpu.VMEM((16,), jnp.int32),
            pltpu.SemaphoreType.DMA,
        ],
    )
    def bcast_kernel(part_hbm, ign_hbm, out_hbm, pbuf, histbuf, ignv, sem):
        c = lax.axis_index("c")
        s = lax.axis_index("s")
        pltpu.sync_copy(ign_hbm, ignv)

        def chunk_body(k, carry):
            kb = k * CH_B
            for g in range(TOKEN_WAYS):
                pltpu.async_copy(
                    part_hbm.at[pl.ds(g * VOCAB_PAD + kb, CH_B)],
                    pbuf.at[pl.ds(g * CH_B, CH_B)],
                    sem,
                )
            for g in range(TOKEN_WAYS):
                pltpu.make_async_copy(
                    part_hbm.at[pl.ds(g * VOCAB_PAD + kb, CH_B)],
                    pbuf.at[pl.ds(g * CH_B, CH_B)],
                    sem,
                ).wait()

            def sum_body(i, carry2):
                acc = pbuf[pl.ds(i * 16, 16)]
                for g in range(1, TOKEN_WAYS):
                    acc = acc + pbuf[pl.ds(g * CH_B + i * 16, 16)]
                histbuf[pl.ds(kb + i * 16, 16)] = acc
                return carry2

            lax.fori_loop(0, CH_B // 16, sum_body, 0)
            return carry

        lax.fori_loop(0, NCH_B, chunk_body, 0)

        iota = lax.iota(jnp.int32, 16)
        idx = jnp.where(iota == 0, ignv[...], iota)
        plsc.store_scatter(histbuf, [idx], jnp.zeros((16,), jnp.float32),
                           mask=iota < 3)

        row0 = c * (BATCH // 2) + s * ROWS_PER_TILE
        for r in range(ROWS_PER_TILE):
            pltpu.sync_copy(
                histbuf.at[pl.ds(0, VOCAB)],
                out_hbm.at[pl.ds((row0 + r) * VOCAB, VOCAB)],
            )

    return bcast_kernel(partials, ign16)


def kernel(src, ignore_index):
    src_flat = src.reshape(-1)
    partials = _sc_histogram(src_flat)
    ign16 = jnp.full((16,), ignore_index, jnp.int32)
    out_flat = _sc_broadcast(partials, ign16)
    return out_flat.reshape(BATCH, VOCAB)


# speedup vs baseline: 2.2816x; 2.2816x over previous
import functools

import jax
import jax.numpy as jnp
from jax import lax
from jax.experimental import pallas as pl
from jax.experimental.pallas import tpu as pltpu
from jax.experimental.pallas import tpu_sc as plsc

VOCAB = 100000
SEQ_LEN = 200
BATCH = 1024
NTOK = SEQ_LEN * BATCH

TOKEN_WAYS = 8
VOCAB_WAYS = 4
VOCAB_PAD = 102400
BINS_PER_TILE = VOCAB_PAD // VOCAB_WAYS
TOK_PER_TILE = NTOK // TOKEN_WAYS

CHUNK = 6400
NCHUNK = TOK_PER_TILE // CHUNK
VREGS_PER_CHUNK = CHUNK // 16

RB = 16
NSTEP = BATCH // RB


def _sc_histogram(src_flat):
    mesh = plsc.VectorSubcoreMesh(core_axis_name="c", subcore_axis_name="s")

    @functools.partial(
        pl.kernel,
        mesh=mesh,
        out_type=jax.ShapeDtypeStruct((TOKEN_WAYS * VOCAB_PAD,), jnp.float32),
        compiler_params=pltpu.CompilerParams(needs_layout_passes=False),
        scratch_types=[
            pltpu.VMEM((CHUNK,), jnp.int32),
            pltpu.VMEM((CHUNK,), jnp.int32),
            pltpu.VMEM((BINS_PER_TILE,), jnp.float32),
            pltpu.SemaphoreType.DMA,
            pltpu.SemaphoreType.DMA,
        ],
    )
    def hist_kernel(src_hbm, out_hbm, buf0, buf1, hist, sem0, sem1):
        c = lax.axis_index("c")
        s = lax.axis_index("s")
        wid = s * 2 + c
        g = wid // VOCAB_WAYS
        v = wid % VOCAB_WAYS
        base = v * BINS_PER_TILE
        tok0 = g * TOK_PER_TILE

        zeros16 = jnp.zeros((16,), jnp.float32)

        def zero_body(i, carry):
            hist[pl.ds(i * 16, 16)] = zeros16
            return carry

        lax.fori_loop(0, BINS_PER_TILE // 16, zero_body, 0)

        ones16 = jnp.ones((16,), jnp.float32)
        bufs = (buf0, buf1)
        sems = (sem0, sem1)

        copies = [None, None]
        copies[0] = pltpu.async_copy(
            src_hbm.at[pl.ds(tok0, CHUNK)], buf0, sem0)
        for ci in range(NCHUNK):
            if ci + 1 < NCHUNK:
                copies[(ci + 1) % 2] = pltpu.async_copy(
                    src_hbm.at[pl.ds(tok0 + (ci + 1) * CHUNK, CHUNK)],
                    bufs[(ci + 1) % 2],
                    sems[(ci + 1) % 2],
                )
            copies[ci % 2].wait()
            buf = bufs[ci % 2]

            def body(i, carry):
                tok = buf[pl.ds(i * 16, 16)]
                rel = tok - base
                mask = (rel >= 0) & (rel < BINS_PER_TILE)
                plsc.addupdate_scatter(hist, [rel], ones16, mask=mask)
                return carry

            lax.fori_loop(0, VREGS_PER_CHUNK, body, 0)

        pltpu.sync_copy(
            hist, out_hbm.at[pl.ds(g * VOCAB_PAD + base, BINS_PER_TILE)])

    return hist_kernel(src_flat).reshape(TOKEN_WAYS, VOCAB_PAD)


DEPTH = 8


def _tc_broadcast(hist_parts, ign):
    def body(ign_ref, hist_ref, out_ref, srcbuf, sems):
        summed = jnp.sum(hist_ref[...], axis=0, keepdims=True)
        cols = lax.broadcasted_iota(jnp.int32, (1, VOCAB_PAD), 1)
        ign_v = ign_ref[0]
        keep = (cols == ign_v) | (cols == 1) | (cols == 2)
        row = jnp.where(keep, 0.0, summed)
        srcbuf[...] = jnp.broadcast_to(row[:, :VOCAB], (RB, VOCAB))

        copies = []
        for j in range(NSTEP):
            cp = pltpu.make_async_copy(
                srcbuf, out_ref.at[pl.ds(j * RB, RB), :], sems.at[j % DEPTH])
            if j >= DEPTH:
                copies[j - DEPTH].wait()
            cp.start()
            copies.append(cp)
        for j in range(NSTEP - DEPTH, NSTEP):
            copies[j].wait()

    return pl.pallas_call(
        body,
        in_specs=[
            pl.BlockSpec(memory_space=pltpu.SMEM),
            pl.BlockSpec(memory_space=pltpu.VMEM),
        ],
        out_specs=pl.BlockSpec(memory_space=pl.ANY),
        out_shape=jax.ShapeDtypeStruct((BATCH, VOCAB), jnp.float32),
        scratch_shapes=[
            pltpu.VMEM((RB, VOCAB), jnp.float32),
            pltpu.SemaphoreType.DMA((DEPTH,)),
        ],
    )(ign, hist_parts)


def kernel(src, ignore_index):
    src_flat = src.reshape(-1)
    hist_parts = _sc_histogram(src_flat)
    ign = jnp.asarray(ignore_index, jnp.int32).reshape(1)
    return _tc_broadcast(hist_parts, ign)
